# baseline (speedup 1.0000x reference)
import jax
import jax.numpy as jnp
import numpy as np
from jax import lax
from jax.experimental import pallas as pl
from jax.experimental.pallas import tpu as pltpu
from jax.experimental.pallas import tpu_sc as plsc

_NC = 2
_NS = 16
_NW = _NC * _NS
_C = 80
_T = 125


def _spmm_body(src_hbm, dst_hbm, w_hbm, x_hbm, out_hbm,
               acc,
               srcv0, srcv1, dstv0, dstv1, dstS0, dstS1, wv0, wv1,
               rows0, rows1, wrows0, wrows1,
               sg0, sg1, ss0, ss1, si0, si1):
    n = acc.shape[0]
    epw = _T * _C
    c = lax.axis_index("c")
    s = lax.axis_index("s")
    wid = s * _NC + c

    srcv = [srcv0, srcv1]
    dstv = [dstv0, dstv1]
    dstS = [dstS0, dstS1]
    wv = [wv0, wv1]
    rows = [rows0, rows1]
    wrows = [wrows0, wrows1]
    sg = [sg0, sg1]
    ss = [ss0, ss1]
    si = [si0, si1]

    def issue_idx(cnum, b):
        base = wid * epw + cnum * _C
        pltpu.async_copy(src_hbm.at[pl.ds(base, _C)], srcv[b], si[b])
        pltpu.async_copy(dst_hbm.at[pl.ds(base, _C)], dstv[b], si[b])
        pltpu.async_copy(w_hbm.at[pl.ds(base, _C)], wv[b], si[b])

    def wait_idx(b):
        pltpu.make_async_copy(src_hbm.at[pl.ds(0, _C)], srcv[b], si[b]).wait()
        pltpu.make_async_copy(dst_hbm.at[pl.ds(0, _C)], dstv[b], si[b]).wait()
        pltpu.make_async_copy(w_hbm.at[pl.ds(0, _C)], wv[b], si[b]).wait()

    def issue_gather(b):
        pltpu.async_copy(x_hbm.at[srcv[b]], rows[b], sg[b])

    def wait_gather(b):
        pltpu.make_async_copy(x_hbm.at[srcv[b]], rows[b], sg[b]).wait()

    def issue_scatter(b):
        pltpu.async_copy(wrows[b], acc.at[dstS[b]], ss[b], add=True)

    def wait_scatter(b):
        pltpu.make_async_copy(wrows[b], acc.at[dstS[b]], ss[b]).wait()

    def multiply(b):
        def gloop(g, cc):
            w16 = wv[b][pl.ds(g * 16, 16)]
            for rr in range(16):
                wsp = jnp.broadcast_to(w16[rr], (16,))
                for j in range(8):
                    vec = rows[b][g * 16 + rr, pl.ds(j * 16, 16)]
                    wrows[b][g * 16 + rr, pl.ds(j * 16, 16)] = vec * wsp
            return cc
        lax.fori_loop(0, _C // 16, gloop, 0)

    def do_chunk(cnum, pb, first=False, gather_next=True, idx_next=True):
        wait_gather(pb)
        if gather_next:
            wait_idx(1 - pb)
            issue_gather(1 - pb)
        if not first:
            wait_scatter(pb)
        multiply(pb)
        for i in range(_C // 16):
            dstS[pb][pl.ds(i * 16, 16)] = dstv[pb][pl.ds(i * 16, 16)]
        issue_scatter(pb)
        if idx_next:
            issue_idx(cnum + 2, pb)

    issue_idx(0, 0)
    issue_idx(1, 1)
    zvec = jnp.zeros((16,), jnp.float32)

    def zrow(i, carry):
        def zcol(j, cc):
            wrows0[i, pl.ds(j * 16, 16)] = zvec
            return cc
        return lax.fori_loop(0, 8, zcol, carry)

    lax.fori_loop(0, _C, zrow, 0)

    @pl.when(s < _NS - 1)
    def _():
        for z in range(8):
            pltpu.sync_copy(wrows0, acc.at[pl.ds(s * 640 + z * _C, _C)])

    @pl.when(s == _NS - 1)
    def _():
        for z in range(5):
            pltpu.sync_copy(wrows0, acc.at[pl.ds(9600 + z * _C, _C)])

    wait_idx(0)
    issue_gather(0)
    plsc.subcore_barrier()

    do_chunk(0, 0, first=True)
    do_chunk(1, 1, first=True)
    do_chunk(2, 0)

    def body(g, carry):
        cb = 3 + 2 * g
        do_chunk(cb, 1)
        do_chunk(cb + 1, 0)
        return carry

    lax.fori_loop(0, (_T - 5) // 2, body, 0)

    do_chunk(_T - 2, 1, idx_next=False)
    do_chunk(_T - 1, 0, idx_next=False, gather_next=False)
    wait_scatter(1)
    wait_scatter(0)

    plsc.subcore_barrier()

    @pl.when(s < _NS - 1)
    def _():
        pltpu.sync_copy(acc.at[pl.ds(s * 640, 640)],
                        out_hbm.at[c, pl.ds(s * 640, 640)])

    @pl.when(s == _NS - 1)
    def _():
        pltpu.sync_copy(acc.at[pl.ds(9600, 400)],
                        out_hbm.at[c, pl.ds(9600, 400)])


def _spmm_sc(src, dst, ew, x):
    n, d = x.shape
    run = pl.kernel(
        _spmm_body,
        out_type=jax.ShapeDtypeStruct((_NC, n, d), jnp.float32),
        compiler_params=pltpu.CompilerParams(needs_layout_passes=False),
        mesh=plsc.VectorSubcoreMesh(core_axis_name="c", subcore_axis_name="s"),
        scratch_types=(
            [pltpu.VMEM_SHARED((n, d), jnp.float32)]
            + [pltpu.VMEM((_C,), jnp.int32) for _ in range(6)]
            + [pltpu.VMEM((_C,), jnp.float32) for _ in range(2)]
            + [pltpu.VMEM((_C, d), jnp.float32) for _ in range(4)]
            + [pltpu.SemaphoreType.DMA for _ in range(6)]
        ),
    )
    return run(src, dst, ew, x)


def _finish_body(a_ref, wl_ref, ws_ref, o_ref):
    ax = a_ref[0] + a_ref[1]
    yl = jnp.dot(ax, wl_ref[...], preferred_element_type=jnp.float32)
    ys = jnp.dot(ax, ws_ref[...], preferred_element_type=jnp.float32)
    o_ref[...] = (jnp.where(yl >= 0, yl, 0.01 * yl)
                  + jnp.where(ys >= 0, ys, 0.01 * ys))


def _finish(parts, wl, ws):
    _, n, d = parts.shape
    return pl.pallas_call(
        _finish_body,
        out_shape=jax.ShapeDtypeStruct((n, d), jnp.float32),
    )(parts, wl, ws)


def _gru_cell(Q, Zt, Wz, Uz, bz, Wr, Ur, br, Wh, Uh, bh):
    upd = jax.nn.sigmoid(Wz @ Zt + Uz @ Q + bz)
    rst = jax.nn.sigmoid(Wr @ Zt + Ur @ Q + br)
    hcap = jnp.tanh(Wh @ Zt + Uh @ (rst * Q) + bh)
    return (1.0 - upd) * Q + upd * hcap


def kernel(node_feats, edge_index, edge_weight, mask,
           long_W0, long_p, long_Wz, long_Uz, long_bz, long_Wr, long_Ur,
           long_br, long_Wh, long_Uh, long_bh,
           short_W0, short_p, short_Wz, short_Uz, short_bz, short_Wr,
           short_Ur, short_br, short_Wh, short_Uh, short_bh):
    t_, n, d = node_feats.shape
    short = 3
    nl = jnp.linalg.norm(long_p) + 1e-12
    ns = jnp.linalg.norm(short_p) + 1e-12
    rows_t = np.concatenate([np.arange(t_), np.arange(t_ - short, t_)])
    A = jnp.stack([node_feats[t] @ long_p / nl + mask[t] for t in range(t_)]
                  + [node_feats[t] @ short_p / ns + mask[t]
                     for t in range(t_ - short, t_)])
    vals, idx = lax.top_k(A, d)
    flat = node_feats.reshape(t_ * n, d)
    gathered = flat[(idx + rows_t[:, None] * n).reshape(-1)]
    Zts = (gathered.reshape(-1, d, d)
           * jnp.tanh(vals)[:, :, None]).transpose(0, 2, 1)

    Q = long_W0
    for t in range(t_):
        Q = _gru_cell(Q, Zts[t], long_Wz, long_Uz, long_bz, long_Wr,
                      long_Ur, long_br, long_Wh, long_Uh, long_bh)
    Wl = Q
    Q = short_W0
    for j in range(short):
        Q = _gru_cell(Q, Zts[t_ + j], short_Wz, short_Uz, short_bz,
                      short_Wr, short_Ur, short_br, short_Wh, short_Uh,
                      short_bh)
    Ws = Q

    src = edge_index[-1, 0]
    dst = edge_index[-1, 1]
    parts = _spmm_sc(src, dst, edge_weight[-1], node_feats[-1])
    return _finish(parts, Wl, Ws)
